# baseline (speedup 1.0000x reference)
import functools

import jax
import jax.numpy as jnp
from jax import lax
from jax.experimental import pallas as pl
from jax.experimental.pallas import tpu as pltpu
from jax.experimental.pallas import tpu_sc as plsc

B = 64
H = 2048
V = 32000
K = 8
VT = 3200
NT = V // VT
NEG_INF = float("-inf")


def _pack_cols(cols, n, dtype):
    io = lax.broadcasted_iota(jnp.int32, (B, n), 1)
    acc = jnp.zeros((B, n), dtype)
    for j, c in enumerate(cols):
        acc = jnp.where(io == j, c.astype(dtype), acc)
    return acc


def _topk_body(h_ref, w_ref, vals_out, idx_out, m_ref, s_ref, rv_ref, ri_ref):
    i = pl.program_id(0)

    @pl.when(i == 0)
    def _init():
        m_ref[...] = jnp.full((B, 128), NEG_INF, jnp.float32)
        s_ref[...] = jnp.zeros((B, 128), jnp.float32)
        rv_ref[...] = jnp.full((B, 128), NEG_INF, jnp.float32)
        ri_ref[...] = jnp.zeros((B, 128), jnp.int32)

    h = h_ref[...]
    w = w_ref[...]
    logits = lax.dot_general(
        h, w, (((1,), (1,)), ((), ())),
        preferred_element_type=jnp.float32)

    tmax = jnp.max(logits, axis=1, keepdims=True)
    m_old = m_ref[:, 0:1]
    s_old = s_ref[:, 0:1]
    m_new = jnp.maximum(m_old, tmax)
    s_new = s_old * jnp.exp(m_old - m_new) + jnp.sum(
        jnp.exp(logits - m_new), axis=1, keepdims=True)
    m_ref[...] = jnp.broadcast_to(m_new, (B, 128))
    s_ref[...] = jnp.broadcast_to(s_new, (B, 128))

    io = lax.broadcasted_iota(jnp.int32, (B, VT), 1)
    x = logits
    base = i * VT
    for j in range(K):
        mx = jnp.max(x, axis=1, keepdims=True)
        loc = jnp.min(jnp.where(x == mx, io, V), axis=1, keepdims=True)
        rv_ref[:, K + j:K + j + 1] = mx
        ri_ref[:, K + j:K + j + 1] = loc + base
        x = jnp.where(io == loc, NEG_INF, x)

    comb_v = rv_ref[:, 0:2 * K]
    comb_i = ri_ref[:, 0:2 * K]
    io2 = lax.broadcasted_iota(jnp.int32, (B, 2 * K), 1)
    new_v, new_i = [], []
    for j in range(K):
        mx = jnp.max(comb_v, axis=1, keepdims=True)
        loc = jnp.min(jnp.where(comb_v == mx, io2, 2 * K), axis=1,
                      keepdims=True)
        sel = io2 == loc
        gidx = jnp.max(jnp.where(sel, comb_i, -1), axis=1, keepdims=True)
        new_v.append(mx)
        new_i.append(gidx)
        comb_v = jnp.where(sel, NEG_INF, comb_v)
    rv_ref[:, 0:K] = _pack_cols(new_v, K, jnp.float32)
    ri_ref[:, 0:K] = _pack_cols(new_i, K, jnp.int32)

    @pl.when(i == NT - 1)
    def _finish():
        lse = m_ref[:, 0:1] + jnp.log(s_ref[:, 0:1])
        vals_out[...] = rv_ref[:, 0:K] - lse
        idx_out[...] = ri_ref[:, 0:K]


def _topk_tc(hidden, w_lm):
    return pl.pallas_call(
        _topk_body,
        grid=(NT,),
        in_specs=[
            pl.BlockSpec((B, H), lambda i: (0, 0)),
            pl.BlockSpec((VT, H), lambda i: (i, 0)),
        ],
        out_specs=[
            pl.BlockSpec((B, K), lambda i: (0, 0)),
            pl.BlockSpec((B, K), lambda i: (0, 0)),
        ],
        out_shape=[
            jax.ShapeDtypeStruct((B, K), jnp.float32),
            jax.ShapeDtypeStruct((B, K), jnp.int32),
        ],
        scratch_shapes=[
            pltpu.VMEM((B, 128), jnp.float32),
            pltpu.VMEM((B, 128), jnp.float32),
            pltpu.VMEM((B, 128), jnp.float32),
            pltpu.VMEM((B, 128), jnp.int32),
        ],
        compiler_params=pltpu.CompilerParams(
            dimension_semantics=("arbitrary",)),
    )(hidden, w_lm)


VP = 32768
WCHUNK = VP // 32


def _premap_sc(d2t_pad):
    mesh = plsc.VectorSubcoreMesh(core_axis_name="c", subcore_axis_name="s")

    @functools.partial(
        pl.kernel,
        mesh=mesh,
        out_type=jax.ShapeDtypeStruct((VP,), jnp.int32),
        scratch_types=[pltpu.VMEM((WCHUNK,), jnp.int32)],
    )
    def k(d2t_hbm, out_hbm, buf):
        wid = lax.axis_index("s") * 2 + lax.axis_index("c")
        base = wid * WCHUNK
        pltpu.sync_copy(d2t_hbm.at[pl.ds(base, WCHUNK)], buf)
        lane = lax.iota(jnp.int32, 16)
        for j in range(WCHUNK // 16):
            t = buf[pl.ds(j * 16, 16)]
            buf[pl.ds(j * 16, 16)] = t + (base + j * 16) + lane
        pltpu.sync_copy(buf, out_hbm.at[pl.ds(base, WCHUNK)])

    return k(d2t_pad)


def _gather_body(idx_ref, pm_ref, out_ref):
    idxv = idx_ref[...]
    pm = pm_ref[...].astype(jnp.float32)
    nr = VP // 128
    ior = lax.broadcasted_iota(jnp.int32, (B, nr), 1)
    iol = lax.broadcasted_iota(jnp.int32, (B, 128), 1)
    cols = []
    for k in range(K):
        g = idxv[:, k:k + 1]
        r = jnp.right_shift(g, 7)
        l = jnp.bitwise_and(g, 127)
        oh = jnp.where(ior == r, 1.0, 0.0).astype(jnp.float32)
        p = lax.dot_general(oh, pm, (((1,), (0,)), ((), ())),
                            precision=lax.Precision.HIGHEST,
                            preferred_element_type=jnp.float32)
        val = jnp.sum(jnp.where(iol == l, p, 0.0), axis=1, keepdims=True)
        cols.append(val)
    out_ref[...] = _pack_cols(cols, K, jnp.float32).astype(jnp.int32)


def _gather_tc(topk_index, premap2d):
    return pl.pallas_call(
        _gather_body,
        in_specs=[
            pl.BlockSpec((B, K), lambda: (0, 0)),
            pl.BlockSpec((VP // 128, 128), lambda: (0, 0)),
        ],
        out_specs=pl.BlockSpec((B, K), lambda: (0, 0)),
        out_shape=jax.ShapeDtypeStruct((B, K), jnp.int32),
    )(topk_index, premap2d)


def kernel(hidden_states, d2t, W_lm):
    d2t_pad = jnp.pad(d2t, (0, VP - V))
    premap = _premap_sc(d2t_pad)
    scores, topk_index = _topk_tc(hidden_states, W_lm)
    mapped = _gather_tc(topk_index, premap.reshape(VP // 128, 128))
    return mapped, scores


# speedup vs baseline: 1.0077x; 1.0077x over previous
import functools

import jax
import jax.numpy as jnp
from jax import lax
from jax.experimental import pallas as pl
from jax.experimental.pallas import tpu as pltpu
from jax.experimental.pallas import tpu_sc as plsc

B = 64
H = 2048
V = 32000
K = 8
VT = 3200
NT = V // VT
NEG_INF = float("-inf")


def _pack_cols(cols, n, dtype):
    io = lax.broadcasted_iota(jnp.int32, (B, n), 1)
    acc = jnp.zeros((B, n), dtype)
    for j, c in enumerate(cols):
        acc = jnp.where(io == j, c.astype(dtype), acc)
    return acc


def _topk_body(h_ref, w_ref, vals_out, idx_out, m_ref, s_ref, rv_ref, ri_ref):
    i = pl.program_id(0)

    @pl.when(i == 0)
    def _init():
        m_ref[...] = jnp.full((B, 128), NEG_INF, jnp.float32)
        s_ref[...] = jnp.zeros((B, 128), jnp.float32)
        rv_ref[...] = jnp.full((B, 128), NEG_INF, jnp.float32)
        ri_ref[...] = jnp.zeros((B, 128), jnp.int32)

    h = h_ref[...]
    w = w_ref[...]
    logits = lax.dot_general(
        h, w, (((1,), (1,)), ((), ())),
        preferred_element_type=jnp.float32)

    tmax = jnp.max(logits, axis=1, keepdims=True)
    m_old = m_ref[:, 0:1]
    s_old = s_ref[:, 0:1]
    m_new = jnp.maximum(m_old, tmax)
    s_new = s_old * jnp.exp(m_old - m_new) + jnp.sum(
        jnp.exp(logits - m_new), axis=1, keepdims=True)
    m_ref[...] = jnp.broadcast_to(m_new, (B, 128))
    s_ref[...] = jnp.broadcast_to(s_new, (B, 128))

    io = lax.broadcasted_iota(jnp.int32, (B, VT), 1)
    x = logits
    base = i * VT
    for j in range(K):
        mx = jnp.max(x, axis=1, keepdims=True)
        loc = jnp.min(jnp.where(x == mx, io, V), axis=1, keepdims=True)
        rv_ref[:, K + j:K + j + 1] = mx
        ri_ref[:, K + j:K + j + 1] = loc + base
        x = jnp.where(io == loc, NEG_INF, x)

    comb_v = rv_ref[:, 0:2 * K]
    comb_i = ri_ref[:, 0:2 * K]
    io2 = lax.broadcasted_iota(jnp.int32, (B, 2 * K), 1)
    new_v, new_i = [], []
    for j in range(K):
        mx = jnp.max(comb_v, axis=1, keepdims=True)
        loc = jnp.min(jnp.where(comb_v == mx, io2, 2 * K), axis=1,
                      keepdims=True)
        sel = io2 == loc
        gidx = jnp.max(jnp.where(sel, comb_i, -1), axis=1, keepdims=True)
        new_v.append(mx)
        new_i.append(gidx)
        comb_v = jnp.where(sel, NEG_INF, comb_v)
    rv_ref[:, 0:K] = _pack_cols(new_v, K, jnp.float32)
    ri_ref[:, 0:K] = _pack_cols(new_i, K, jnp.int32)

    @pl.when(i == NT - 1)
    def _finish():
        lse = m_ref[:, 0:1] + jnp.log(s_ref[:, 0:1])
        vals_out[...] = rv_ref[:, 0:K] - lse
        idx_out[...] = ri_ref[:, 0:K]


def _topk_tc(hidden, w_lm):
    return pl.pallas_call(
        _topk_body,
        grid=(NT,),
        in_specs=[
            pl.BlockSpec((B, H), lambda i: (0, 0)),
            pl.BlockSpec((VT, H), lambda i: (i, 0)),
        ],
        out_specs=[
            pl.BlockSpec((B, K), lambda i: (0, 0)),
            pl.BlockSpec((B, K), lambda i: (0, 0)),
        ],
        out_shape=[
            jax.ShapeDtypeStruct((B, K), jnp.float32),
            jax.ShapeDtypeStruct((B, K), jnp.int32),
        ],
        scratch_shapes=[
            pltpu.VMEM((B, 128), jnp.float32),
            pltpu.VMEM((B, 128), jnp.float32),
            pltpu.VMEM((B, 128), jnp.float32),
            pltpu.VMEM((B, 128), jnp.int32),
        ],
        compiler_params=pltpu.CompilerParams(
            dimension_semantics=("arbitrary",)),
    )(hidden, w_lm)


WCHUNK = V // 32


def _premap_sc(d2t):
    mesh = plsc.VectorSubcoreMesh(core_axis_name="c", subcore_axis_name="s")

    @functools.partial(
        pl.kernel,
        mesh=mesh,
        out_type=jax.ShapeDtypeStruct((V,), jnp.int32),
        scratch_types=[pltpu.VMEM((WCHUNK,), jnp.int32)],
    )
    def k(d2t_hbm, out_hbm, buf):
        wid = lax.axis_index("s") * 2 + lax.axis_index("c")
        base = wid * WCHUNK
        pltpu.sync_copy(d2t_hbm.at[pl.ds(base, WCHUNK)], buf)
        lane = lax.iota(jnp.int32, 16)
        offs = [j * 16 for j in range(WCHUNK // 16)] + [WCHUNK - 16]
        for off in offs:
            t = buf[pl.ds(off, 16)]
            buf[pl.ds(off, 16)] = t + (base + off) + lane
        pltpu.sync_copy(buf, out_hbm.at[pl.ds(base, WCHUNK)])

    return k(d2t)


def _gather_body(idx_ref, pm_ref, out_ref):
    idxv = idx_ref[...]
    pm = pm_ref[...].astype(jnp.float32)
    nr = V // 128
    ior = lax.broadcasted_iota(jnp.int32, (B, nr), 1)
    iol = lax.broadcasted_iota(jnp.int32, (B, 128), 1)
    cols = []
    for k in range(K):
        g = idxv[:, k:k + 1]
        r = jnp.right_shift(g, 7)
        l = jnp.bitwise_and(g, 127)
        oh = jnp.where(ior == r, 1.0, 0.0).astype(jnp.float32)
        p = lax.dot_general(oh, pm, (((1,), (0,)), ((), ())),
                            precision=lax.Precision.HIGHEST,
                            preferred_element_type=jnp.float32)
        val = jnp.sum(jnp.where(iol == l, p, 0.0), axis=1, keepdims=True)
        cols.append(val)
    out_ref[...] = _pack_cols(cols, K, jnp.float32).astype(jnp.int32)


def _gather_tc(topk_index, premap2d):
    return pl.pallas_call(
        _gather_body,
        in_specs=[
            pl.BlockSpec((B, K), lambda: (0, 0)),
            pl.BlockSpec((V // 128, 128), lambda: (0, 0)),
        ],
        out_specs=pl.BlockSpec((B, K), lambda: (0, 0)),
        out_shape=jax.ShapeDtypeStruct((B, K), jnp.int32),
    )(topk_index, premap2d)


def kernel(hidden_states, d2t, W_lm):
    premap = _premap_sc(d2t)
    scores, topk_index = _topk_tc(hidden_states, W_lm)
    mapped = _gather_tc(topk_index, premap.reshape(V // 128, 128))
    return mapped, scores


# speedup vs baseline: 1.0102x; 1.0024x over previous
import functools

import jax
import jax.numpy as jnp
from jax import lax
from jax.experimental import pallas as pl
from jax.experimental.pallas import tpu as pltpu
from jax.experimental.pallas import tpu_sc as plsc

B = 64
H = 2048
V = 32000
K = 8
VT = 3200
NT = V // VT
NEG_INF = float("-inf")


def _pack_cols(cols, n, dtype):
    io = lax.broadcasted_iota(jnp.int32, (B, n), 1)
    acc = jnp.zeros((B, n), dtype)
    for j, c in enumerate(cols):
        acc = jnp.where(io == j, c.astype(dtype), acc)
    return acc


def _topk_body(h_ref, w_ref, vals_out, idx_out, m_ref, s_ref, rv_ref, ri_ref):
    i = pl.program_id(0)

    @pl.when(i == 0)
    def _init():
        m_ref[...] = jnp.full((B, 128), NEG_INF, jnp.float32)
        s_ref[...] = jnp.zeros((B, 128), jnp.float32)
        rv_ref[...] = jnp.full((B, 128), NEG_INF, jnp.float32)
        ri_ref[...] = jnp.zeros((B, 128), jnp.int32)

    h = h_ref[...]
    w = w_ref[...]
    logits = lax.dot_general(
        h, w, (((1,), (1,)), ((), ())),
        preferred_element_type=jnp.float32)

    tmax = jnp.max(logits, axis=1, keepdims=True)
    m_old = m_ref[:, 0:1]
    s_old = s_ref[:, 0:1]
    m_new = jnp.maximum(m_old, tmax)
    s_new = s_old * jnp.exp(m_old - m_new) + jnp.sum(
        jnp.exp(logits - m_new), axis=1, keepdims=True)
    m_ref[...] = jnp.broadcast_to(m_new, (B, 128))
    s_ref[...] = jnp.broadcast_to(s_new, (B, 128))

    io = lax.broadcasted_iota(jnp.int32, (B, VT), 1)
    x = logits
    base = i * VT
    for j in range(K):
        mx = jnp.max(x, axis=1, keepdims=True)
        loc = jnp.min(jnp.where(x == mx, io, V), axis=1, keepdims=True)
        rv_ref[:, K + j:K + j + 1] = mx
        ri_ref[:, K + j:K + j + 1] = loc + base
        x = jnp.where(io == loc, NEG_INF, x)

    comb_v = rv_ref[:, 0:2 * K]
    comb_i = ri_ref[:, 0:2 * K]
    io2 = lax.broadcasted_iota(jnp.int32, (B, 2 * K), 1)
    new_v, new_i = [], []
    for j in range(K):
        mx = jnp.max(comb_v, axis=1, keepdims=True)
        loc = jnp.min(jnp.where(comb_v == mx, io2, 2 * K), axis=1,
                      keepdims=True)
        sel = io2 == loc
        gidx = jnp.max(jnp.where(sel, comb_i, -1), axis=1, keepdims=True)
        new_v.append(mx)
        new_i.append(gidx)
        comb_v = jnp.where(sel, NEG_INF, comb_v)
    rv_ref[:, 0:K] = _pack_cols(new_v, K, jnp.float32)
    ri_ref[:, 0:K] = _pack_cols(new_i, K, jnp.int32)

    @pl.when(i == NT - 1)
    def _finish():
        lse = m_ref[:, 0:1] + jnp.log(s_ref[:, 0:1])
        vals_out[...] = rv_ref[:, 0:K] - lse
        idx_out[...] = ri_ref[:, 0:K]


def _topk_tc(hidden, w_lm):
    return pl.pallas_call(
        _topk_body,
        grid=(NT,),
        in_specs=[
            pl.BlockSpec((B, H), lambda i: (0, 0)),
            pl.BlockSpec((VT, H), lambda i: (i, 0)),
        ],
        out_specs=[
            pl.BlockSpec((B, K), lambda i: (0, 0)),
            pl.BlockSpec((B, K), lambda i: (0, 0)),
        ],
        out_shape=[
            jax.ShapeDtypeStruct((B, K), jnp.float32),
            jax.ShapeDtypeStruct((B, K), jnp.int32),
        ],
        scratch_shapes=[
            pltpu.VMEM((B, 128), jnp.float32),
            pltpu.VMEM((B, 128), jnp.float32),
            pltpu.VMEM((B, 128), jnp.float32),
            pltpu.VMEM((B, 128), jnp.int32),
        ],
        compiler_params=pltpu.CompilerParams(
            dimension_semantics=("arbitrary",)),
    )(hidden, w_lm)


WCHUNK = V // 32


def _premap_sc(d2t):
    mesh = plsc.VectorSubcoreMesh(core_axis_name="c", subcore_axis_name="s")

    @functools.partial(
        pl.kernel,
        mesh=mesh,
        out_type=jax.ShapeDtypeStruct((V,), jnp.int32),
        scratch_types=[pltpu.VMEM((WCHUNK,), jnp.int32)],
    )
    def k(d2t_hbm, out_hbm, buf):
        wid = lax.axis_index("s") * 2 + lax.axis_index("c")
        base = wid * WCHUNK
        pltpu.sync_copy(d2t_hbm.at[pl.ds(base, WCHUNK)], buf)
        lane = lax.iota(jnp.int32, 16)
        t_last = buf[pl.ds(WCHUNK - 16, 16)]
        for j in range(WCHUNK // 16):
            t = buf[pl.ds(j * 16, 16)]
            buf[pl.ds(j * 16, 16)] = t + (base + j * 16) + lane
        buf[pl.ds(WCHUNK - 16, 16)] = t_last + (base + WCHUNK - 16) + lane
        pltpu.sync_copy(buf, out_hbm.at[pl.ds(base, WCHUNK)])

    return k(d2t)


def _gather_body(idx_ref, pm_ref, out_ref):
    idxv = idx_ref[...]
    pm = pm_ref[...].astype(jnp.float32)
    nr = V // 128
    ior = lax.broadcasted_iota(jnp.int32, (B, nr), 1)
    iol = lax.broadcasted_iota(jnp.int32, (B, 128), 1)
    cols = []
    for k in range(K):
        g = idxv[:, k:k + 1]
        r = jnp.right_shift(g, 7)
        l = jnp.bitwise_and(g, 127)
        oh = jnp.where(ior == r, 1.0, 0.0).astype(jnp.float32)
        p = lax.dot_general(oh, pm, (((1,), (0,)), ((), ())),
                            precision=lax.Precision.HIGHEST,
                            preferred_element_type=jnp.float32)
        val = jnp.sum(jnp.where(iol == l, p, 0.0), axis=1, keepdims=True)
        cols.append(val)
    out_ref[...] = _pack_cols(cols, K, jnp.float32).astype(jnp.int32)


def _gather_tc(topk_index, premap2d):
    return pl.pallas_call(
        _gather_body,
        in_specs=[
            pl.BlockSpec((B, K), lambda: (0, 0)),
            pl.BlockSpec((V // 128, 128), lambda: (0, 0)),
        ],
        out_specs=pl.BlockSpec((B, K), lambda: (0, 0)),
        out_shape=jax.ShapeDtypeStruct((B, K), jnp.int32),
    )(topk_index, premap2d)


def kernel(hidden_states, d2t, W_lm):
    premap = _premap_sc(d2t)
    scores, topk_index = _topk_tc(hidden_states, W_lm)
    mapped = _gather_tc(topk_index, premap.reshape(V // 128, 128))
    return mapped, scores


# speedup vs baseline: 1.0149x; 1.0046x over previous
import functools

import jax
import jax.numpy as jnp
from jax import lax
from jax.experimental import pallas as pl
from jax.experimental.pallas import tpu as pltpu
from jax.experimental.pallas import tpu_sc as plsc

B = 64
H = 2048
V = 32000
K = 8
VT = 3200
NT = V // VT
NEG_INF = float("-inf")


def _pack_cols(cols, n, dtype):
    io = lax.broadcasted_iota(jnp.int32, (B, n), 1)
    acc = jnp.zeros((B, n), dtype)
    for j, c in enumerate(cols):
        acc = jnp.where(io == j, c.astype(dtype), acc)
    return acc


def _topk_body(h_ref, w_ref, vals_out, idx_out, m_ref, s_ref, rv_ref, ri_ref):
    i = pl.program_id(0)

    @pl.when(i == 0)
    def _init():
        m_ref[...] = jnp.full((B, 128), NEG_INF, jnp.float32)
        s_ref[...] = jnp.zeros((B, 128), jnp.float32)
        rv_ref[...] = jnp.full((B, 128), NEG_INF, jnp.float32)
        ri_ref[...] = jnp.zeros((B, 128), jnp.int32)

    h = h_ref[...]
    w = w_ref[...]
    logits = lax.dot_general(
        h, w, (((1,), (1,)), ((), ())),
        preferred_element_type=jnp.float32)

    tmax = jnp.max(logits, axis=1, keepdims=True)
    m_old = m_ref[:, 0:1]
    s_old = s_ref[:, 0:1]
    m_new = jnp.maximum(m_old, tmax)
    s_new = s_old * jnp.exp(m_old - m_new) + jnp.sum(
        jnp.exp(logits - m_new), axis=1, keepdims=True)
    m_ref[...] = jnp.broadcast_to(m_new, (B, 128))
    s_ref[...] = jnp.broadcast_to(s_new, (B, 128))

    io = lax.broadcasted_iota(jnp.int32, (B, VT), 1)
    x = logits
    base = i * VT
    for j in range(K):
        mx = jnp.max(x, axis=1, keepdims=True)
        loc = jnp.min(jnp.where(x == mx, io, V), axis=1, keepdims=True)
        rv_ref[:, K + j:K + j + 1] = mx
        ri_ref[:, K + j:K + j + 1] = loc + base
        x = jnp.where(io == loc, NEG_INF, x)

    comb_v = rv_ref[:, 0:2 * K]
    comb_i = ri_ref[:, 0:2 * K]
    io2 = lax.broadcasted_iota(jnp.int32, (B, 2 * K), 1)
    new_v, new_i = [], []
    for j in range(K):
        mx = jnp.max(comb_v, axis=1, keepdims=True)
        loc = jnp.min(jnp.where(comb_v == mx, io2, 2 * K), axis=1,
                      keepdims=True)
        sel = io2 == loc
        gidx = jnp.max(jnp.where(sel, comb_i, -1), axis=1, keepdims=True)
        new_v.append(mx)
        new_i.append(gidx)
        comb_v = jnp.where(sel, NEG_INF, comb_v)
    rv_ref[:, 0:K] = _pack_cols(new_v, K, jnp.float32)
    ri_ref[:, 0:K] = _pack_cols(new_i, K, jnp.int32)

    @pl.when(i == NT - 1)
    def _finish():
        lse = m_ref[:, 0:1] + jnp.log(s_ref[:, 0:1])
        vals_out[...] = rv_ref[:, 0:K] - lse
        idx_out[...] = ri_ref[:, 0:K]


def _topk_tc(hidden, w_lm):
    return pl.pallas_call(
        _topk_body,
        grid=(NT,),
        in_specs=[
            pl.BlockSpec((B, H), lambda i: (0, 0)),
            pl.BlockSpec((VT, H), lambda i: (i, 0)),
        ],
        out_specs=[
            pl.BlockSpec((B, K), lambda i: (0, 0)),
            pl.BlockSpec((B, K), lambda i: (0, 0)),
        ],
        out_shape=[
            jax.ShapeDtypeStruct((B, K), jnp.float32),
            jax.ShapeDtypeStruct((B, K), jnp.int32),
        ],
        scratch_shapes=[
            pltpu.VMEM((B, 128), jnp.float32),
            pltpu.VMEM((B, 128), jnp.float32),
            pltpu.VMEM((B, 128), jnp.float32),
            pltpu.VMEM((B, 128), jnp.int32),
        ],
        compiler_params=pltpu.CompilerParams(
            dimension_semantics=("arbitrary",)),
    )(hidden, w_lm)


def _d2t_map_sc(d2t, idx_flat):
    n = idx_flat.shape[0]
    mesh = plsc.VectorSubcoreMesh(core_axis_name="c", subcore_axis_name="s")

    @functools.partial(
        pl.kernel,
        mesh=mesh,
        out_type=jax.ShapeDtypeStruct((n,), jnp.int32),
        scratch_types=[
            pltpu.VMEM((16,), jnp.int32),
            pltpu.VMEM((16,), jnp.int32),
            pltpu.SemaphoreType.DMA,
        ],
    )
    def k(d2t_hbm, idx_hbm, out_hbm, idx_v, g_v, sem):
        wid = lax.axis_index("s") * 2 + lax.axis_index("c")
        base = wid * 16
        pltpu.sync_copy(idx_hbm.at[pl.ds(base, 16)], idx_v)
        pltpu.async_copy(d2t_hbm.at[idx_v], g_v, sem).wait()
        g_v[...] = g_v[...] + idx_v[...]
        pltpu.sync_copy(g_v, out_hbm.at[pl.ds(base, 16)])

    return k(d2t, idx_flat)


def kernel(hidden_states, d2t, W_lm):
    scores, topk_index = _topk_tc(hidden_states, W_lm)
    mapped = _d2t_map_sc(d2t, topk_index.reshape(B * K)).reshape(B, K)
    return mapped, scores
